# baseline (speedup 1.0000x reference)
import functools

import jax
import jax.numpy as jnp
from jax import lax
from jax.experimental import pallas as pl
from jax.experimental.pallas import tpu as pltpu
from jax.experimental.pallas import tpu_sc as plsc

B = 16384
F = 64
H = 100
HP = 128
IDX_CHUNK = 128
RB = 8


def _make_sc_gather(num_cores, num_subcores):
    nw = num_cores * num_subcores
    bpw = B // nw
    nch = bpw // IDX_CHUNK
    mesh = plsc.VectorSubcoreMesh(core_axis_name="c", subcore_axis_name="s")

    @functools.partial(
        pl.kernel,
        mesh=mesh,
        compiler_params=pltpu.CompilerParams(use_tc_tiling_on_sc=False),
        out_type=(
            jax.ShapeDtypeStruct((B, F), jnp.float32),
            jax.ShapeDtypeStruct((B, F), jnp.float32),
            jax.ShapeDtypeStruct((B, RB), jnp.float32),
            jax.ShapeDtypeStruct((B, RB), jnp.float32),
        ),
        scratch_types=[
            pltpu.VMEM((nch, IDX_CHUNK), jnp.int32),
            pltpu.VMEM((nch, IDX_CHUNK), jnp.int32),
            pltpu.VMEM((nch, IDX_CHUNK), jnp.int32),
            pltpu.VMEM((nch, IDX_CHUNK), jnp.int32),
            pltpu.VMEM((bpw, F), jnp.float32),
            pltpu.VMEM((bpw, F), jnp.float32),
            pltpu.VMEM((bpw, RB), jnp.float32),
            pltpu.VMEM((bpw, RB), jnp.float32),
            pltpu.SemaphoreType.DMA,
            pltpu.SemaphoreType.DMA,
            pltpu.SemaphoreType.DMA,
            pltpu.SemaphoreType.DMA,
        ],
    )
    def sc_gather(uidx_hbm, iidx_hbm, uidx8_hbm, iidx8_hbm,
                  uf_hbm, ub8_hbm, if_hbm, ib8_hbm,
                  u_out, it_out, ub_out, ib_out,
                  uidx_v, iidx_v, uidx8_v, iidx8_v,
                  u_v, it_v, ub_v, ib_v,
                  sem_u, sem_i, sem_ub, sem_ib):
        wid = lax.axis_index("s") * num_cores + lax.axis_index("c")
        base = wid * bpw
        pltpu.sync_copy(uidx_hbm.at[pl.ds(wid * nch, nch)], uidx_v)
        pltpu.sync_copy(iidx_hbm.at[pl.ds(wid * nch, nch)], iidx_v)
        pltpu.sync_copy(uidx8_hbm.at[pl.ds(wid * nch, nch)], uidx8_v)
        pltpu.sync_copy(iidx8_hbm.at[pl.ds(wid * nch, nch)], iidx8_v)
        copies = []
        for j in range(nch):
            rows = pl.ds(j * IDX_CHUNK, IDX_CHUNK)
            copies.append(pltpu.async_copy(uf_hbm.at[uidx_v.at[j]], u_v.at[rows], sem_u))
            copies.append(pltpu.async_copy(if_hbm.at[iidx_v.at[j]], it_v.at[rows], sem_i))
            copies.append(pltpu.async_copy(ub8_hbm.at[uidx8_v.at[j]], ub_v.at[rows], sem_ub))
            copies.append(pltpu.async_copy(ib8_hbm.at[iidx8_v.at[j]], ib_v.at[rows], sem_ib))
        for c in copies:
            c.wait()
        pltpu.sync_copy(u_v, u_out.at[pl.ds(base, bpw)])
        pltpu.sync_copy(it_v, it_out.at[pl.ds(base, bpw)])
        pltpu.sync_copy(ub_v, ub_out.at[pl.ds(base, bpw)])
        pltpu.sync_copy(ib_v, ib_out.at[pl.ds(base, bpw)])

    return sc_gather


def _mlp_body(u_ref, it_ref, ub8_ref, ib8_ref, ulane_ref, ilane_ref,
              w1u_ref, w1i_ref, w1ub_ref, w1ib_ref, b1_ref,
              w2_ref, b2_ref, w3_ref, b3_ref, w4_ref, b4_ref,
              sd_ref, out_ref):
    u = u_ref[...]
    it = it_ref[...]
    blk = u.shape[0]
    lane8 = lax.broadcasted_iota(jnp.int32, (blk, RB), 1)
    ub = jnp.sum(jnp.where(lane8 == ulane_ref[...], ub8_ref[...], 0.0),
                 axis=1, keepdims=True)
    ib = jnp.sum(jnp.where(lane8 == ilane_ref[...], ib8_ref[...], 0.0),
                 axis=1, keepdims=True)

    def dot(a, b):
        return lax.dot_general(a, b, (((1,), (0,)), ((), ())),
                               preferred_element_type=jnp.float32,
                               precision=lax.Precision.HIGHEST)

    z1 = (dot(u, w1u_ref[...]) + dot(it, w1i_ref[...])
          + ub * w1ub_ref[...] + ib * w1ib_ref[...] + b1_ref[...])
    x1 = jnp.maximum(z1, 0.0)
    x2 = jnp.maximum(dot(x1, w2_ref[...]) + b2_ref[...], 0.0)
    sd = jnp.sum(u * it, axis=1, keepdims=True) + ub + ib
    x3 = dot(x2, w3_ref[...]) + b3_ref[...] + sd
    o = jnp.sum(x3 * w4_ref[...], axis=1, keepdims=True) + b4_ref[...]
    sd_ref[...] = sd
    out_ref[...] = o


def _mlp_call(u, it, ub8, ib8, ulane, ilane, weights, blk):
    grid = (B // blk,)
    row_spec = lambda w: pl.BlockSpec((blk, w), lambda i: (i, 0))
    full = lambda a: pl.BlockSpec(a.shape, lambda i: (0,) * a.ndim)
    return pl.pallas_call(
        _mlp_body,
        grid=grid,
        in_specs=[row_spec(F), row_spec(F), row_spec(RB), row_spec(RB),
                  row_spec(1), row_spec(1)]
                 + [full(w) for w in weights],
        out_specs=[pl.BlockSpec((blk, 1), lambda i: (i, 0)),
                   pl.BlockSpec((blk, 1), lambda i: (i, 0))],
        out_shape=[jax.ShapeDtypeStruct((B, 1), jnp.float32),
                   jax.ShapeDtypeStruct((B, 1), jnp.float32)],
        compiler_params=pltpu.CompilerParams(
            dimension_semantics=("arbitrary",)),
    )(u, it, ub8, ib8, ulane, ilane, *weights)


def kernel(item_in, user_in, user_factors, user_bias, item_factors, item_bias,
           W1, b1, W2, b2, W3, b3, W4, b4):
    info = plsc.get_sparse_core_info()
    uidx = user_in.reshape(B // IDX_CHUNK, IDX_CHUNK)
    iidx = item_in.reshape(B // IDX_CHUNK, IDX_CHUNK)
    uidx8 = uidx // RB
    iidx8 = iidx // RB
    ulane = user_in % RB
    ilane = item_in % RB
    ub_tab = user_bias.reshape(user_bias.shape[0] // RB, RB)
    ib_tab = item_bias.reshape(item_bias.shape[0] // RB, RB)

    sc_gather = _make_sc_gather(info.num_cores, info.num_subcores)
    u, it, ub8, ib8 = sc_gather(uidx, iidx, uidx8, iidx8,
                                user_factors, ub_tab, item_factors, ib_tab)

    w1u = jnp.zeros((F, HP), jnp.float32).at[:, :H].set(W1[0:F])
    w1ub = jnp.zeros((1, HP), jnp.float32).at[:, :H].set(W1[F:F + 1])
    w1i = jnp.zeros((F, HP), jnp.float32).at[:, :H].set(W1[F + 1:2 * F + 1])
    w1ib = jnp.zeros((1, HP), jnp.float32).at[:, :H].set(W1[2 * F + 1:2 * F + 2])
    b1p = jnp.zeros((1, HP), jnp.float32).at[:, :H].set(b1[None, :])
    w2p = jnp.zeros((HP, HP), jnp.float32).at[:H, :H].set(W2)
    b2p = jnp.zeros((1, HP), jnp.float32).at[:, :H].set(b2[None, :])
    w3p = jnp.zeros((HP, HP), jnp.float32).at[:H, :H].set(W3)
    b3p = jnp.zeros((1, HP), jnp.float32).at[:, :H].set(b3[None, :])
    w4p = jnp.zeros((1, HP), jnp.float32).at[:, :H].set(W4[:, 0][None, :])
    b4p = b4.reshape(1, 1)

    weights = (w1u, w1i, w1ub, w1ib, b1p, w2p, b2p, w3p, b3p, w4p, b4p)
    sd, out = _mlp_call(u, it, ub8, ib8, ulane, ilane, weights, blk=2048)
    return sd, out


# speedup vs baseline: 1.3036x; 1.3036x over previous
import functools

import jax
import jax.numpy as jnp
from jax import lax
from jax.experimental import pallas as pl
from jax.experimental.pallas import tpu as pltpu
from jax.experimental.pallas import tpu_sc as plsc

B = 16384
F = 64
IDX_CHUNK = 128
RB = 8


def _make_sc_gather(num_cores, num_subcores):
    nw = num_cores * num_subcores
    bpw = B // nw
    nch = bpw // IDX_CHUNK
    mesh = plsc.VectorSubcoreMesh(core_axis_name="c", subcore_axis_name="s")

    @functools.partial(
        pl.kernel,
        mesh=mesh,
        compiler_params=pltpu.CompilerParams(use_tc_tiling_on_sc=False),
        out_type=(
            jax.ShapeDtypeStruct((B, 2 * F), jnp.float32),
            jax.ShapeDtypeStruct((B, 2 * RB), jnp.float32),
        ),
        scratch_types=[
            pltpu.VMEM((nch, IDX_CHUNK), jnp.int32),
            pltpu.VMEM((nch, IDX_CHUNK), jnp.int32),
            pltpu.VMEM((nch, IDX_CHUNK), jnp.int32),
            pltpu.VMEM((nch, IDX_CHUNK), jnp.int32),
            pltpu.VMEM((bpw, F), jnp.float32),
            pltpu.VMEM((bpw, F), jnp.float32),
            pltpu.VMEM((bpw, RB), jnp.float32),
            pltpu.VMEM((bpw, RB), jnp.float32),
            pltpu.SemaphoreType.DMA,
            pltpu.SemaphoreType.DMA,
            pltpu.SemaphoreType.DMA,
            pltpu.SemaphoreType.DMA,
        ],
    )
    def sc_gather(uidx_hbm, iidx_hbm, uidx8_hbm, iidx8_hbm,
                  uf_hbm, ub8_hbm, if_hbm, ib8_hbm,
                  feat_out, b16_out,
                  uidx_v, iidx_v, uidx8_v, iidx8_v,
                  u_v, it_v, ub_v, ib_v,
                  sem_u, sem_i, sem_ub, sem_ib):
        wid = lax.axis_index("s") * num_cores + lax.axis_index("c")
        base = wid * bpw
        pltpu.sync_copy(uidx_hbm.at[pl.ds(wid * nch, nch)], uidx_v)
        pltpu.sync_copy(iidx_hbm.at[pl.ds(wid * nch, nch)], iidx_v)
        pltpu.sync_copy(uidx8_hbm.at[pl.ds(wid * nch, nch)], uidx8_v)
        pltpu.sync_copy(iidx8_hbm.at[pl.ds(wid * nch, nch)], iidx8_v)
        copies = []
        for j in range(nch):
            rows = pl.ds(j * IDX_CHUNK, IDX_CHUNK)
            copies.append(pltpu.async_copy(uf_hbm.at[uidx_v.at[j]], u_v.at[rows], sem_u))
            copies.append(pltpu.async_copy(if_hbm.at[iidx_v.at[j]], it_v.at[rows], sem_i))
            copies.append(pltpu.async_copy(ub8_hbm.at[uidx8_v.at[j]], ub_v.at[rows], sem_ub))
            copies.append(pltpu.async_copy(ib8_hbm.at[iidx8_v.at[j]], ib_v.at[rows], sem_ib))
        for c in copies:
            c.wait()
        rows = pl.ds(base, bpw)
        pltpu.sync_copy(u_v, feat_out.at[rows, pl.ds(0, F)])
        pltpu.sync_copy(it_v, feat_out.at[rows, pl.ds(F, F)])
        pltpu.sync_copy(ub_v, b16_out.at[rows, pl.ds(0, RB)])
        pltpu.sync_copy(ib_v, b16_out.at[rows, pl.ds(RB, RB)])

    return sc_gather


def _mlp_body(feat_ref, b16_ref, ui_ref, ii_ref,
              w1_ref, b1_ref, w2_ref, b2_ref, w3_ref, b3_ref, w4_ref, b4_ref,
              sd_ref, out_ref):
    feat = feat_ref[...]
    blk = feat.shape[0]
    u = feat[:, :F]
    it = feat[:, F:]
    b16 = b16_ref[...]
    ulane = ui_ref[...] & (RB - 1)
    ilane = ii_ref[...] & (RB - 1)
    lane = lax.broadcasted_iota(jnp.int32, (blk, RB), 1)
    ub = jnp.sum(jnp.where(lane == ulane, b16[:, :RB], 0.0),
                 axis=1, keepdims=True)
    ib = jnp.sum(jnp.where(lane == ilane, b16[:, RB:], 0.0),
                 axis=1, keepdims=True)

    def dot(a, b):
        return lax.dot_general(a, b, (((1,), (0,)), ((), ())),
                               preferred_element_type=jnp.float32)

    w1 = w1_ref[...]
    z1 = (dot(u, w1[0:F]) + dot(it, w1[F + 1:2 * F + 1])
          + ub * w1[F:F + 1] + ib * w1[2 * F + 1:2 * F + 2]
          + b1_ref[...][None, :])
    x1 = jnp.maximum(z1, 0.0)
    x2 = jnp.maximum(dot(x1, w2_ref[...]) + b2_ref[...][None, :], 0.0)
    sd = jnp.sum(u * it, axis=1, keepdims=True) + ub + ib
    x3 = dot(x2, w3_ref[...]) + b3_ref[...][None, :] + sd
    o = dot(x3, w4_ref[...]) + b4_ref[...][None, :]
    sd_ref[...] = sd
    out_ref[...] = o


def _mlp_call(feat, b16, user_in, item_in, weights, blk):
    grid = (B // blk,)
    row_spec = lambda w: pl.BlockSpec((blk, w), lambda i: (i, 0))
    full = lambda a: pl.BlockSpec(a.shape, lambda i: (0,) * a.ndim)
    return pl.pallas_call(
        _mlp_body,
        grid=grid,
        in_specs=[row_spec(2 * F), row_spec(2 * RB), row_spec(1), row_spec(1)]
                 + [full(w) for w in weights],
        out_specs=[pl.BlockSpec((blk, 1), lambda i: (i, 0)),
                   pl.BlockSpec((blk, 1), lambda i: (i, 0))],
        out_shape=[jax.ShapeDtypeStruct((B, 1), jnp.float32),
                   jax.ShapeDtypeStruct((B, 1), jnp.float32)],
        compiler_params=pltpu.CompilerParams(
            dimension_semantics=("arbitrary",)),
    )(feat, b16, user_in, item_in, *weights)


def kernel(item_in, user_in, user_factors, user_bias, item_factors, item_bias,
           W1, b1, W2, b2, W3, b3, W4, b4):
    info = plsc.get_sparse_core_info()
    uidx = user_in.reshape(B // IDX_CHUNK, IDX_CHUNK)
    iidx = item_in.reshape(B // IDX_CHUNK, IDX_CHUNK)
    uidx8 = uidx // RB
    iidx8 = iidx // RB
    ub_tab = user_bias.reshape(user_bias.shape[0] // RB, RB)
    ib_tab = item_bias.reshape(item_bias.shape[0] // RB, RB)

    sc_gather = _make_sc_gather(info.num_cores, info.num_subcores)
    feat, b16 = sc_gather(uidx, iidx, uidx8, iidx8,
                          user_factors, ub_tab, item_factors, ib_tab)
    weights = (W1, b1, W2, b2, W3, b3, W4, b4)
    sd, out = _mlp_call(feat, b16, user_in, item_in, weights, blk=2048)
    return sd, out


# speedup vs baseline: 1.5001x; 1.1507x over previous
import functools

import jax
import jax.numpy as jnp
from jax import lax
from jax.experimental import pallas as pl
from jax.experimental.pallas import tpu as pltpu
from jax.experimental.pallas import tpu_sc as plsc

B = 16384
F = 64
IDX_CHUNK = 128
RB = 8
L = 16


def _make_sc_gather(num_cores, num_subcores):
    nw = num_cores * num_subcores
    bpw = B // nw
    nch = bpw // IDX_CHUNK
    mesh = plsc.VectorSubcoreMesh(core_axis_name="c", subcore_axis_name="s")

    @functools.partial(
        pl.kernel,
        mesh=mesh,
        compiler_params=pltpu.CompilerParams(use_tc_tiling_on_sc=False,
                                             needs_layout_passes=False),
        out_type=(
            jax.ShapeDtypeStruct((B, 2 * F), jnp.float32),
            jax.ShapeDtypeStruct((B, RB), jnp.float32),
        ),
        scratch_types=[
            pltpu.VMEM((nch, IDX_CHUNK), jnp.int32),
            pltpu.VMEM((nch, IDX_CHUNK), jnp.int32),
            pltpu.VMEM((nch, IDX_CHUNK), jnp.int32),
            pltpu.VMEM((nch, IDX_CHUNK), jnp.int32),
            pltpu.VMEM((bpw, F), jnp.float32),
            pltpu.VMEM((bpw, F), jnp.float32),
            pltpu.VMEM((bpw, RB), jnp.float32),
            pltpu.VMEM((bpw, RB), jnp.float32),
            pltpu.VMEM((bpw, RB), jnp.float32),
            pltpu.SemaphoreType.DMA,
            pltpu.SemaphoreType.DMA,
            pltpu.SemaphoreType.DMA,
            pltpu.SemaphoreType.DMA,
        ],
    )
    def sc_gather(uidx_hbm, iidx_hbm, uidx8_hbm, iidx8_hbm,
                  uf_hbm, ub8_hbm, if_hbm, ib8_hbm,
                  feat_out, b8_out,
                  uidx_v, iidx_v, uidx8_v, iidx8_v,
                  u_v, it_v, ub_v, ib_v, b8_v,
                  sem_u, sem_i, sem_ub, sem_ib):
        wid = lax.axis_index("s") * num_cores + lax.axis_index("c")
        base = wid * bpw
        pltpu.sync_copy(uidx_hbm.at[pl.ds(wid * nch, nch)], uidx_v)
        pltpu.sync_copy(iidx_hbm.at[pl.ds(wid * nch, nch)], iidx_v)
        pltpu.sync_copy(uidx8_hbm.at[pl.ds(wid * nch, nch)], uidx8_v)
        pltpu.sync_copy(iidx8_hbm.at[pl.ds(wid * nch, nch)], iidx8_v)
        copies = []
        for j in range(nch):
            rows = pl.ds(j * IDX_CHUNK, IDX_CHUNK)
            copies.append(pltpu.async_copy(uf_hbm.at[uidx_v.at[j]], u_v.at[rows], sem_u))
            copies.append(pltpu.async_copy(if_hbm.at[iidx_v.at[j]], it_v.at[rows], sem_i))
            copies.append(pltpu.async_copy(ub8_hbm.at[uidx8_v.at[j]], ub_v.at[rows], sem_ub))
            copies.append(pltpu.async_copy(ib8_hbm.at[iidx8_v.at[j]], ib_v.at[rows], sem_ib))
        for c in copies:
            c.wait()
        for g in range(bpw // L):
            rows16 = lax.iota(jnp.int32, L) + g * L
            sl = pl.ds((g % (IDX_CHUNK // L)) * L, L)
            uidx16 = uidx_v[g // (IDX_CHUNK // L), sl]
            iidx16 = iidx_v[g // (IDX_CHUNK // L), sl]
            ubv = plsc.load_gather(ub_v, [rows16, uidx16 & (RB - 1)])
            ibv = plsc.load_gather(ib_v, [rows16, iidx16 & (RB - 1)])
            plsc.store_scatter(b8_v, [rows16, jnp.zeros((L,), jnp.int32)], ubv)
            plsc.store_scatter(b8_v, [rows16, jnp.ones((L,), jnp.int32)], ibv)
        rows = pl.ds(base, bpw)
        pltpu.sync_copy(u_v, feat_out.at[rows, pl.ds(0, F)])
        pltpu.sync_copy(it_v, feat_out.at[rows, pl.ds(F, F)])
        pltpu.sync_copy(b8_v, b8_out.at[rows])

    return sc_gather


def _mlp_body(feat_ref, b8_ref, w1cat_ref, w1ub_ref, w1ib_ref, b1_ref,
              w2_ref, b2_ref, w3_ref, b3_ref, w4_ref, b4_ref,
              sd_ref, out_ref):
    feat = feat_ref[...]
    b8 = b8_ref[...]
    ub = b8[:, 0:1]
    ib = b8[:, 1:2]

    def dot(a, b):
        return lax.dot_general(a, b, (((1,), (0,)), ((), ())),
                               preferred_element_type=jnp.float32)

    z1 = (dot(feat, w1cat_ref[...]) + ub * w1ub_ref[...] + ib * w1ib_ref[...]
          + b1_ref[...][None, :])
    x1 = jnp.maximum(z1, 0.0)
    x2 = jnp.maximum(dot(x1, w2_ref[...]) + b2_ref[...][None, :], 0.0)
    sd = jnp.sum(feat[:, :F] * feat[:, F:], axis=1, keepdims=True) + ub + ib
    x3 = dot(x2, w3_ref[...]) + b3_ref[...][None, :] + sd
    o = dot(x3, w4_ref[...]) + b4_ref[...][None, :]
    sd_ref[...] = sd
    out_ref[...] = o


def _mlp_call(feat, b8, weights, blk):
    grid = (B // blk,)
    row_spec = lambda w: pl.BlockSpec((blk, w), lambda i: (i, 0))
    full = lambda a: pl.BlockSpec(a.shape, lambda i: (0,) * a.ndim)
    return pl.pallas_call(
        _mlp_body,
        grid=grid,
        in_specs=[row_spec(2 * F), row_spec(RB)] + [full(w) for w in weights],
        out_specs=[pl.BlockSpec((blk, 1), lambda i: (i, 0)),
                   pl.BlockSpec((blk, 1), lambda i: (i, 0))],
        out_shape=[jax.ShapeDtypeStruct((B, 1), jnp.float32),
                   jax.ShapeDtypeStruct((B, 1), jnp.float32)],
        compiler_params=pltpu.CompilerParams(
            dimension_semantics=("arbitrary",)),
    )(feat, b8, *weights)


def kernel(item_in, user_in, user_factors, user_bias, item_factors, item_bias,
           W1, b1, W2, b2, W3, b3, W4, b4):
    info = plsc.get_sparse_core_info()
    uidx = user_in.reshape(B // IDX_CHUNK, IDX_CHUNK)
    iidx = item_in.reshape(B // IDX_CHUNK, IDX_CHUNK)
    uidx8 = uidx // RB
    iidx8 = iidx // RB
    ub_tab = user_bias.reshape(user_bias.shape[0] // RB, RB)
    ib_tab = item_bias.reshape(item_bias.shape[0] // RB, RB)

    sc_gather = _make_sc_gather(info.num_cores, info.num_subcores)
    feat, b8 = sc_gather(uidx, iidx, uidx8, iidx8,
                         user_factors, ub_tab, item_factors, ib_tab)

    HP = 128
    H = W2.shape[0]
    w1cat = jnp.zeros((2 * F, HP), jnp.float32).at[:, :H].set(
        jnp.concatenate([W1[0:F], W1[F + 1:2 * F + 1]], axis=0))
    w1ub = jnp.zeros((1, HP), jnp.float32).at[:, :H].set(W1[F:F + 1])
    w1ib = jnp.zeros((1, HP), jnp.float32).at[:, :H].set(W1[2 * F + 1:2 * F + 2])
    b1p = jnp.zeros((HP,), jnp.float32).at[:H].set(b1)
    w2p = jnp.zeros((HP, HP), jnp.float32).at[:H, :H].set(W2)
    b2p = jnp.zeros((HP,), jnp.float32).at[:H].set(b2)
    w3p = jnp.zeros((HP, HP), jnp.float32).at[:H, :H].set(W3)
    b3p = jnp.zeros((HP,), jnp.float32).at[:H].set(b3)
    w4p = jnp.zeros((HP, 1), jnp.float32).at[:H].set(W4)
    weights = (w1cat, w1ub, w1ib, b1p, w2p, b2p, w3p, b3p, w4p, b4)
    sd, out = _mlp_call(feat, b8, weights, blk=2048)
    return sd, out


# speedup vs baseline: 1.5257x; 1.0171x over previous
import functools

import jax
import jax.numpy as jnp
from jax import lax
from jax.experimental import pallas as pl
from jax.experimental.pallas import tpu as pltpu
from jax.experimental.pallas import tpu_sc as plsc

B = 16384
F = 64
IDX_CHUNK = 128
RB = 8
L = 16


def _make_sc_factor_gather(num_cores, num_subcores):
    nw = num_cores * num_subcores
    bpw = B // nw
    nch = bpw // IDX_CHUNK
    mesh = plsc.VectorSubcoreMesh(core_axis_name="c", subcore_axis_name="s")

    @functools.partial(
        pl.kernel,
        mesh=mesh,
        compiler_params=pltpu.CompilerParams(use_tc_tiling_on_sc=False,
                                             needs_layout_passes=False),
        out_type=jax.ShapeDtypeStruct((B, 2 * F), jnp.float32),
        scratch_types=[
            pltpu.VMEM((nch, IDX_CHUNK), jnp.int32),
            pltpu.VMEM((nch, IDX_CHUNK), jnp.int32),
            pltpu.VMEM((bpw, F), jnp.float32),
            pltpu.VMEM((bpw, F), jnp.float32),
            pltpu.SemaphoreType.DMA,
            pltpu.SemaphoreType.DMA,
        ],
    )
    def sc_factors(uidx_hbm, iidx_hbm, uf_hbm, if_hbm, feat_out,
                   uidx_v, iidx_v, u_v, it_v, sem_u, sem_i):
        wid = lax.axis_index("s") * num_cores + lax.axis_index("c")
        base = wid * bpw
        pltpu.sync_copy(uidx_hbm.at[pl.ds(wid * nch, nch)], uidx_v)
        pltpu.sync_copy(iidx_hbm.at[pl.ds(wid * nch, nch)], iidx_v)
        copies = []
        for j in range(nch):
            rows = pl.ds(j * IDX_CHUNK, IDX_CHUNK)
            copies.append(pltpu.async_copy(uf_hbm.at[uidx_v.at[j]], u_v.at[rows], sem_u))
            copies.append(pltpu.async_copy(if_hbm.at[iidx_v.at[j]], it_v.at[rows], sem_i))
        for c in copies:
            c.wait()
        rows = pl.ds(base, bpw)
        pltpu.sync_copy(u_v, feat_out.at[rows, pl.ds(0, F)])
        pltpu.sync_copy(it_v, feat_out.at[rows, pl.ds(F, F)])

    return sc_factors


def _make_sc_bias_gather(num_cores, num_subcores):
    nw = num_cores * num_subcores
    bpw = B // nw
    nch = bpw // IDX_CHUNK
    mesh = plsc.VectorSubcoreMesh(core_axis_name="c", subcore_axis_name="s")

    @functools.partial(
        pl.kernel,
        mesh=mesh,
        compiler_params=pltpu.CompilerParams(use_tc_tiling_on_sc=False,
                                             needs_layout_passes=False),
        out_type=jax.ShapeDtypeStruct((B, RB), jnp.float32),
        scratch_types=[
            pltpu.VMEM((nch, IDX_CHUNK), jnp.int32),
            pltpu.VMEM((nch, IDX_CHUNK), jnp.int32),
            pltpu.VMEM((nch, IDX_CHUNK), jnp.int32),
            pltpu.VMEM((nch, IDX_CHUNK), jnp.int32),
            pltpu.VMEM((bpw, RB), jnp.float32),
            pltpu.VMEM((bpw, RB), jnp.float32),
            pltpu.VMEM((bpw, RB), jnp.float32),
            pltpu.SemaphoreType.DMA,
            pltpu.SemaphoreType.DMA,
        ],
    )
    def sc_bias(uidx_hbm, iidx_hbm, ub8_hbm, ib8_hbm, b8_out,
                uidx_v, iidx_v, uidx8_v, iidx8_v, ub_v, ib_v, b8_v,
                sem_ub, sem_ib):
        wid = lax.axis_index("s") * num_cores + lax.axis_index("c")
        base = wid * bpw
        pltpu.sync_copy(uidx_hbm.at[pl.ds(wid * nch, nch)], uidx_v)
        pltpu.sync_copy(iidx_hbm.at[pl.ds(wid * nch, nch)], iidx_v)
        for j in range(nch):
            for m in range(IDX_CHUNK // L):
                s = pl.ds(m * L, L)
                uidx8_v[j, s] = uidx_v[j, s] >> 3
                iidx8_v[j, s] = iidx_v[j, s] >> 3
        copies = []
        for j in range(nch):
            rows = pl.ds(j * IDX_CHUNK, IDX_CHUNK)
            copies.append(pltpu.async_copy(ub8_hbm.at[uidx8_v.at[j]], ub_v.at[rows], sem_ub))
            copies.append(pltpu.async_copy(ib8_hbm.at[iidx8_v.at[j]], ib_v.at[rows], sem_ib))
        for c in copies:
            c.wait()
        for g in range(bpw // L):
            rows16 = lax.iota(jnp.int32, L) + g * L
            sl = pl.ds((g % (IDX_CHUNK // L)) * L, L)
            uidx16 = uidx_v[g // (IDX_CHUNK // L), sl]
            iidx16 = iidx_v[g // (IDX_CHUNK // L), sl]
            ubv = plsc.load_gather(ub_v, [rows16, uidx16 & (RB - 1)])
            ibv = plsc.load_gather(ib_v, [rows16, iidx16 & (RB - 1)])
            plsc.store_scatter(b8_v, [rows16, jnp.zeros((L,), jnp.int32)], ubv)
            plsc.store_scatter(b8_v, [rows16, jnp.ones((L,), jnp.int32)], ibv)
        pltpu.sync_copy(b8_v, b8_out.at[pl.ds(base, bpw)])

    return sc_bias


def _mlp_body(feat_ref, b8_ref, w1cat_ref, w1ub_ref, w1ib_ref, b1_ref,
              w2_ref, b2_ref, w3_ref, b3_ref, w4_ref, b4_ref,
              sd_ref, out_ref):
    feat = feat_ref[...]
    b8 = b8_ref[...]
    ub = b8[:, 0:1]
    ib = b8[:, 1:2]

    def dot(a, b):
        return lax.dot_general(a, b, (((1,), (0,)), ((), ())),
                               preferred_element_type=jnp.float32)

    z1 = (dot(feat, w1cat_ref[...]) + ub * w1ub_ref[...] + ib * w1ib_ref[...]
          + b1_ref[...][None, :])
    x1 = jnp.maximum(z1, 0.0)
    x2 = jnp.maximum(dot(x1, w2_ref[...]) + b2_ref[...][None, :], 0.0)
    sd = jnp.sum(feat[:, :F] * feat[:, F:], axis=1, keepdims=True) + ub + ib
    x3 = dot(x2, w3_ref[...]) + b3_ref[...][None, :] + sd
    o = dot(x3, w4_ref[...]) + b4_ref[...][None, :]
    sd_ref[...] = sd
    out_ref[...] = o


def _mlp_call(feat, b8, weights, blk):
    grid = (B // blk,)
    row_spec = lambda w: pl.BlockSpec((blk, w), lambda i: (i, 0))
    full = lambda a: pl.BlockSpec(a.shape, lambda i: (0,) * a.ndim)
    return pl.pallas_call(
        _mlp_body,
        grid=grid,
        in_specs=[row_spec(2 * F), row_spec(RB)] + [full(w) for w in weights],
        out_specs=[pl.BlockSpec((blk, 1), lambda i: (i, 0)),
                   pl.BlockSpec((blk, 1), lambda i: (i, 0))],
        out_shape=[jax.ShapeDtypeStruct((B, 1), jnp.float32),
                   jax.ShapeDtypeStruct((B, 1), jnp.float32)],
        compiler_params=pltpu.CompilerParams(
            dimension_semantics=("arbitrary",)),
    )(feat, b8, *weights)


def kernel(item_in, user_in, user_factors, user_bias, item_factors, item_bias,
           W1, b1, W2, b2, W3, b3, W4, b4):
    info = plsc.get_sparse_core_info()
    uidx = user_in.reshape(B // IDX_CHUNK, IDX_CHUNK)
    iidx = item_in.reshape(B // IDX_CHUNK, IDX_CHUNK)
    ub_tab = user_bias.reshape(user_bias.shape[0] // RB, RB)
    ib_tab = item_bias.reshape(item_bias.shape[0] // RB, RB)

    sc_factors = _make_sc_factor_gather(info.num_cores, info.num_subcores)
    sc_bias = _make_sc_bias_gather(info.num_cores, info.num_subcores)
    feat = sc_factors(uidx, iidx, user_factors, item_factors)
    b8 = sc_bias(uidx, iidx, ub_tab, ib_tab)

    HP = 128
    H = W2.shape[0]
    w1cat = jnp.zeros((2 * F, HP), jnp.float32).at[:, :H].set(
        jnp.concatenate([W1[0:F], W1[F + 1:2 * F + 1]], axis=0))
    w1ub = jnp.zeros((1, HP), jnp.float32).at[:, :H].set(W1[F:F + 1])
    w1ib = jnp.zeros((1, HP), jnp.float32).at[:, :H].set(W1[2 * F + 1:2 * F + 2])
    b1p = jnp.zeros((HP,), jnp.float32).at[:H].set(b1)
    w2p = jnp.zeros((HP, HP), jnp.float32).at[:H, :H].set(W2)
    b2p = jnp.zeros((HP,), jnp.float32).at[:H].set(b2)
    w3p = jnp.zeros((HP, HP), jnp.float32).at[:H, :H].set(W3)
    b3p = jnp.zeros((HP,), jnp.float32).at[:H].set(b3)
    w4p = jnp.zeros((HP, 1), jnp.float32).at[:H].set(W4)
    weights = (w1cat, w1ub, w1ib, b1p, w2p, b2p, w3p, b3p, w4p, b4)
    sd, out = _mlp_call(feat, b8, weights, blk=4096)
    return sd, out
